# baseline (speedup 1.0000x reference)
import functools

import jax
import jax.numpy as jnp
from jax import lax
from jax.experimental import pallas as pl
from jax.experimental.pallas import tpu as pltpu
from jax.experimental.pallas import tpu_sc as plsc

_N = 1_000_000
_HIST = 200
_HIST_PAD = 208
_NCHUNK = _HIST_PAD // 16
_NW = 32
_CHUNK = 31_264
_LAST = _N - 31 * _CHUNK

_mesh = plsc.VectorSubcoreMesh(core_axis_name="c", subcore_axis_name="s")


@functools.partial(
    pl.kernel,
    out_type=(
        jax.ShapeDtypeStruct((_N,), jnp.float32),
        jax.ShapeDtypeStruct((_N,), jnp.float32),
    ),
    mesh=_mesh,
    scratch_types=[
        pltpu.VMEM((_HIST_PAD,), jnp.int32),
        pltpu.VMEM((_HIST_PAD,), jnp.float32),
        pltpu.VMEM((_HIST_PAD,), jnp.int32),
        pltpu.VMEM((_HIST_PAD,), jnp.float32),
        pltpu.VMEM((_CHUNK,), jnp.float32),
        pltpu.VMEM((_CHUNK,), jnp.float32),
    ],
    compiler_params=pltpu.CompilerParams(needs_layout_passes=False),
)
def _flat_input_sc(item_h, rating_h, titem_h, trating_h, out0_h, out1_h,
                   idx_v, val_v, tidx_v, tval_v, buf0, buf1):
    c = lax.axis_index("c")
    s = lax.axis_index("s")
    wid = s * 2 + c
    base = wid * _CHUNK
    end = jnp.minimum(base + _CHUNK, _N)
    nvec = (end - base) // 16

    pltpu.sync_copy(item_h, idx_v)
    pltpu.sync_copy(rating_h, val_v)
    pltpu.sync_copy(titem_h, tidx_v)
    pltpu.sync_copy(trating_h, tval_v)

    zeros = jnp.zeros((16,), jnp.float32)
    nans = jnp.full((16,), jnp.nan, jnp.float32)

    def fill_body(i, carry):
        o = i * 64
        buf0[pl.ds(o, 16)] = zeros
        buf0[pl.ds(o + 16, 16)] = zeros
        buf0[pl.ds(o + 32, 16)] = zeros
        buf0[pl.ds(o + 48, 16)] = zeros
        buf1[pl.ds(o, 16)] = nans
        buf1[pl.ds(o + 16, 16)] = nans
        buf1[pl.ds(o + 32, 16)] = nans
        buf1[pl.ds(o + 48, 16)] = nans
        return carry

    nv4 = nvec // 4
    lax.fori_loop(0, nv4, fill_body, None)

    def fill_tail(i, carry):
        o = i * 16
        buf0[pl.ds(o, 16)] = zeros
        buf1[pl.ds(o, 16)] = nans
        return carry

    lax.fori_loop(nv4 * 4, nvec, fill_tail, None)

    lane = lax.iota(jnp.int32, 16)

    def scatter_all(idxref, valref, buf):
        for k in range(_NCHUNK):
            idx = idxref[pl.ds(k * 16, 16)]
            val = valref[pl.ds(k * 16, 16)]
            inr = (idx >= base) & (idx < end)
            if (k + 1) * 16 > _HIST:
                inr = inr & (lane < (_HIST - k * 16))
            local = jnp.clip(idx - base, 0, _CHUNK - 1)
            for l in range(16):
                plsc.store_scatter(buf, [local], val, mask=inr & (lane == l))

    scatter_all(idx_v, val_v, buf0)
    scatter_all(tidx_v, tval_v, buf1)

    @pl.when(wid < _NW - 1)
    def _():
        pltpu.sync_copy(buf0.at[pl.ds(0, _CHUNK)], out0_h.at[pl.ds(base, _CHUNK)])
        pltpu.sync_copy(buf1.at[pl.ds(0, _CHUNK)], out1_h.at[pl.ds(base, _CHUNK)])

    @pl.when(wid == _NW - 1)
    def _():
        pltpu.sync_copy(buf0.at[pl.ds(0, _LAST)], out0_h.at[pl.ds(base, _LAST)])
        pltpu.sync_copy(buf1.at[pl.ds(0, _LAST)], out1_h.at[pl.ds(base, _LAST)])


@jax.jit
def kernel(item, rating, target_item, target_rating):
    pad = _HIST_PAD - _HIST
    item_p = jnp.pad(item.astype(jnp.int32), (0, pad))
    rating_p = jnp.pad(rating, (0, pad))
    titem_p = jnp.pad(target_item.astype(jnp.int32), (0, pad))
    trating_p = jnp.pad(target_rating, (0, pad))
    return _flat_input_sc(item_p, rating_p, titem_p, trating_p)


# speedup vs baseline: 2.5735x; 2.5735x over previous
import jax
import jax.numpy as jnp
from jax import lax
from jax.experimental import pallas as pl
from jax.experimental.pallas import tpu as pltpu

_N = 1_000_000
_HIST = 200
_NMAIN = 999_936
_NTAIL = _N - _NMAIN
_W = 1024
_PAD = 1_000_448
_FILL = 8192
_NFULL = _PAD // _FILL
_FTAIL = _PAD - _NFULL * _FILL


def _body(item_s, rating_s, titem_s, trating_s, out0_h, out1_h,
          buf0, buf1, tl0, tl1, sem0, sem1, sem2, sem3):
    lanes = lax.broadcasted_iota(jnp.int32, (_W,), 0)
    lanes64 = lax.broadcasted_iota(jnp.int32, (_NTAIL,), 0)

    def build(buf, tl, idx_s, val_s, const):
        vec = jnp.full((_FILL,), const, jnp.float32)

        def step(i, carry):
            buf[pl.ds(i * _FILL, _FILL)] = vec
            return carry

        lax.fori_loop(0, _NFULL, step, None)
        buf[pl.ds(_NFULL * _FILL, _FTAIL)] = vec[:_FTAIL]
        tl[...] = jnp.full((_NTAIL,), const, jnp.float32)

        for j in range(_HIST):
            idx = idx_s[j]
            val = val_s[j]

            @pl.when(idx < _NMAIN)
            def _():
                base = pl.multiple_of(idx & ~(_W - 1), _W)
                w = buf[pl.ds(base, _W)]
                buf[pl.ds(base, _W)] = jnp.where(lanes == idx - base, val, w)

            @pl.when(idx >= _NMAIN)
            def _():
                tl[...] = jnp.where(lanes64 == idx - _NMAIN, val, tl[...])

    build(buf0, tl0, item_s, rating_s, 0.0)
    cp0 = pltpu.make_async_copy(buf0.at[pl.ds(0, _NMAIN)],
                                out0_h.at[pl.ds(0, _NMAIN)], sem0)
    cp0.start()
    cp0t = pltpu.make_async_copy(tl0, out0_h.at[pl.ds(_NMAIN, _NTAIL)], sem2)
    cp0t.start()

    build(buf1, tl1, titem_s, trating_s, float("nan"))
    cp1 = pltpu.make_async_copy(buf1.at[pl.ds(0, _NMAIN)],
                                out1_h.at[pl.ds(0, _NMAIN)], sem1)
    cp1.start()
    cp1t = pltpu.make_async_copy(tl1, out1_h.at[pl.ds(_NMAIN, _NTAIL)], sem3)
    cp1t.start()

    cp0.wait()
    cp0t.wait()
    cp1.wait()
    cp1t.wait()


_flat_input_tc = pl.pallas_call(
    _body,
    in_specs=[pl.BlockSpec(memory_space=pltpu.SMEM)] * 4,
    out_specs=[pl.BlockSpec(memory_space=pl.ANY)] * 2,
    out_shape=[
        jax.ShapeDtypeStruct((_N,), jnp.float32),
        jax.ShapeDtypeStruct((_N,), jnp.float32),
    ],
    scratch_shapes=[
        pltpu.VMEM((_PAD,), jnp.float32),
        pltpu.VMEM((_PAD,), jnp.float32),
        pltpu.VMEM((_NTAIL,), jnp.float32),
        pltpu.VMEM((_NTAIL,), jnp.float32),
        pltpu.SemaphoreType.DMA,
        pltpu.SemaphoreType.DMA,
        pltpu.SemaphoreType.DMA,
        pltpu.SemaphoreType.DMA,
    ],
)


@jax.jit
def kernel(item, rating, target_item, target_rating):
    return _flat_input_tc(item.astype(jnp.int32), rating,
                          target_item.astype(jnp.int32), target_rating)


# speedup vs baseline: 4.1799x; 1.6242x over previous
import jax
import jax.numpy as jnp
from jax import lax
from jax.experimental import pallas as pl
from jax.experimental.pallas import tpu as pltpu

_N = 1_000_000
_HIST = 200
_NMAIN = 999_936
_NTAIL = _N - _NMAIN
_HALF = 499_968
_W = 1024
_PAD = 1_000_448
_FILL = 8192
_NFULL = _PAD // _FILL
_FTAIL = _PAD - _NFULL * _FILL
_G = 4


def _body(item_s, rating_s, titem_s, trating_s, out0_h, out1_h,
          buf0, buf1, tl0, tl1, *sems):
    lanes = lax.broadcasted_iota(jnp.int32, (_W,), 0)

    def build(buf, tl, idx_s, val_s, const):
        vec = jnp.full((_FILL,), const, jnp.float32)

        def step(i, carry):
            buf[pl.ds(i * _FILL, _FILL)] = vec
            return carry

        lax.fori_loop(0, _NFULL, step, None)
        buf[pl.ds(_NFULL * _FILL, _FTAIL)] = vec[:_FTAIL]

        for g in range(0, _HIST, _G):
            idx = [idx_s[g + i] for i in range(_G)]
            val = [val_s[g + i] for i in range(_G)]
            base = [pl.multiple_of(ix & ~(_W - 1), _W) for ix in idx]
            w = [buf[pl.ds(b, _W)] for b in base]
            new = []
            for i in range(_G):
                wi = w[i]
                for k in range(i):
                    wi = jnp.where(base[i] == base[k], new[k], wi)
                new.append(jnp.where(lanes == idx[i] - base[i], val[i], wi))
            for i in range(_G):
                buf[pl.ds(base[i], _W)] = new[i]

        tl[...] = buf[pl.ds(_NMAIN, _NTAIL)]

    build(buf0, tl0, item_s, rating_s, 0.0)
    cps0 = [
        pltpu.make_async_copy(buf0.at[pl.ds(0, _HALF)],
                              out0_h.at[pl.ds(0, _HALF)], sems[0]),
        pltpu.make_async_copy(buf0.at[pl.ds(_HALF, _HALF)],
                              out0_h.at[pl.ds(_HALF, _HALF)], sems[1]),
        pltpu.make_async_copy(tl0, out0_h.at[pl.ds(_NMAIN, _NTAIL)], sems[2]),
    ]
    for cp in cps0:
        cp.start()

    build(buf1, tl1, titem_s, trating_s, float("nan"))
    cps1 = [
        pltpu.make_async_copy(buf1.at[pl.ds(0, _HALF)],
                              out1_h.at[pl.ds(0, _HALF)], sems[3]),
        pltpu.make_async_copy(buf1.at[pl.ds(_HALF, _HALF)],
                              out1_h.at[pl.ds(_HALF, _HALF)], sems[4]),
        pltpu.make_async_copy(tl1, out1_h.at[pl.ds(_NMAIN, _NTAIL)], sems[5]),
    ]
    for cp in cps1:
        cp.start()

    for cp in cps0 + cps1:
        cp.wait()


_flat_input_tc = pl.pallas_call(
    _body,
    in_specs=[pl.BlockSpec(memory_space=pltpu.SMEM)] * 4,
    out_specs=[pl.BlockSpec(memory_space=pl.ANY)] * 2,
    out_shape=[
        jax.ShapeDtypeStruct((_N,), jnp.float32),
        jax.ShapeDtypeStruct((_N,), jnp.float32),
    ],
    scratch_shapes=[
        pltpu.VMEM((_PAD,), jnp.float32),
        pltpu.VMEM((_PAD,), jnp.float32),
        pltpu.VMEM((_NTAIL,), jnp.float32),
        pltpu.VMEM((_NTAIL,), jnp.float32),
    ] + [pltpu.SemaphoreType.DMA] * 6,
)


@jax.jit
def kernel(item, rating, target_item, target_rating):
    return _flat_input_tc(item.astype(jnp.int32), rating,
                          target_item.astype(jnp.int32), target_rating)
